# baseline (speedup 1.0000x reference)
import functools

import jax
import jax.numpy as jnp
from jax import lax
from jax.experimental import pallas as pl
from jax.experimental.pallas import tpu as pltpu
from jax.experimental.pallas import tpu_sc as plsc

_N_PROMPTS = 2000
_N_CTX = 16
_CTX_DIM = 512
_SEQ = 77
_SUFFIX_LEN = _SEQ - 1 - _N_CTX
_NC = 2
_NS = 16
_NW = _NC * _NS
_LANES = 16
_CHUNKS = _CTX_DIM // _LANES


def kernel(ctx, token_prefix, token_suffix):
    mesh = plsc.VectorSubcoreMesh(core_axis_name="c", subcore_axis_name="s")

    @functools.partial(
        pl.kernel,
        out_type=jax.ShapeDtypeStruct((_N_PROMPTS, _SEQ, _CTX_DIM),
                                      jnp.float32),
        mesh=mesh,
        scratch_types=[
            pltpu.VMEM((_N_CTX, _CTX_DIM), jnp.float32),
            pltpu.VMEM((1, _CTX_DIM), jnp.float32),
            pltpu.VMEM((_SUFFIX_LEN, _CTX_DIM), jnp.float32),
            pltpu.VMEM((_SEQ, _CTX_DIM), jnp.float32),
            pltpu.VMEM((_SEQ, _CTX_DIM), jnp.float32),
            pltpu.SemaphoreType.DMA,
            pltpu.SemaphoreType.DMA,
            pltpu.SemaphoreType.DMA,
        ],
    )
    def _sc(ctx_hbm, pre_hbm, suf_hbm, out_hbm,
            ctxv, pbuf, sufin, blk0, blk1, si, so0, so1):
        w = lax.axis_index("s") * _NC + lax.axis_index("c")
        start = w * 62 + jnp.minimum(w, 16)
        blks = (blk0, blk1)
        sos = (so0, so1)

        pltpu.sync_copy(ctx_hbm, ctxv)
        for r in range(_N_CTX):
            for c in range(_CHUNKS):
                sl = pl.ds(c * _LANES, _LANES)
                v = ctxv[r, sl]
                blk0[1 + r, sl] = v
                blk1[1 + r, sl] = v

        def issue_in(j):
            p = start + j
            pltpu.async_copy(pre_hbm.at[p], pbuf, si)
            pltpu.async_copy(suf_hbm.at[p], sufin, si)

        def wait_in():
            pltpu.make_async_copy(pre_hbm.at[0], pbuf, si).wait()
            pltpu.make_async_copy(suf_hbm.at[0], sufin, si).wait()

        def assemble(b):
            blk = blks[b]
            for c in range(_CHUNKS):
                sl = pl.ds(c * _LANES, _LANES)
                blk[0, sl] = pbuf[0, sl]

            @plsc.parallel_loop(0, _SUFFIX_LEN, unroll=4)
            def _(i):
                for c in range(_CHUNKS):
                    sl = pl.ds(c * _LANES, _LANES)
                    blk[1 + _N_CTX + i, sl] = sufin[i, sl]

        def issue_out(j, b):
            p = start + j
            pltpu.async_copy(blks[b], out_hbm.at[p], sos[b])

        def drain_out(b):
            pltpu.make_async_copy(blks[b], out_hbm.at[0], sos[b]).wait()

        issue_in(0)
        wait_in()
        assemble(0)
        issue_in(1)
        issue_out(0, 0)
        wait_in()
        assemble(1)
        issue_in(2)
        issue_out(1, 1)

        def step(k, carry):
            j = 2 * k
            wait_in()
            drain_out(0)
            assemble(0)
            issue_in(j + 1)
            issue_out(j, 0)

            wait_in()
            drain_out(1)
            assemble(1)

            @pl.when((j + 2 < 62) | (w < 16))
            def _():
                issue_in(j + 2)

            issue_out(j + 1, 1)
            return carry

        lax.fori_loop(1, 31, step, 0)

        @pl.when(w < 16)
        def _():
            wait_in()
            drain_out(0)
            assemble(0)
            issue_out(62, 0)

        drain_out(1)
        drain_out(0)

    return _sc(ctx, token_prefix, token_suffix)


# speedup vs baseline: 1.1365x; 1.1365x over previous
import jax
import jax.numpy as jnp
from jax.experimental import pallas as pl

_N_PROMPTS = 2000
_N_CTX = 16
_CTX_DIM = 512
_SEQ = 77
_SUFFIX_LEN = _SEQ - 1 - _N_CTX

_G = 50


def _body(ctx_ref, pre_ref, suf_ref, out_ref):
    out_ref[:, 0:1, :] = pre_ref[...]
    ctx = ctx_ref[...]
    out_ref[:, 1:1 + _N_CTX, :] = jnp.broadcast_to(
        ctx[None, :, :], (_G, _N_CTX, _CTX_DIM))
    out_ref[:, 1 + _N_CTX:, :] = suf_ref[...]


def kernel(ctx, token_prefix, token_suffix):
    grid = (_N_PROMPTS // _G,)
    return pl.pallas_call(
        _body,
        grid=grid,
        in_specs=[
            pl.BlockSpec((_N_CTX, _CTX_DIM), lambda i: (0, 0)),
            pl.BlockSpec((_G, 1, _CTX_DIM), lambda i: (i, 0, 0)),
            pl.BlockSpec((_G, _SUFFIX_LEN, _CTX_DIM), lambda i: (i, 0, 0)),
        ],
        out_specs=pl.BlockSpec((_G, _SEQ, _CTX_DIM), lambda i: (i, 0, 0)),
        out_shape=jax.ShapeDtypeStruct((_N_PROMPTS, _SEQ, _CTX_DIM),
                                       jnp.float32),
    )(ctx, token_prefix, token_suffix)
